# baseline (speedup 1.0000x reference)
import jax
import jax.numpy as jnp
from jax.experimental import pallas as pl
from jax.experimental.pallas import tpu as pltpu

_MIN_NORM = 1e-15
_EPS = 1e-7


def _lorentz_kernel(x_ref, a_ref, o_ref, xt_ref):
    @pl.when(pl.program_id(0) == 0)
    def _compute_tangent():
        x = x_ref[...]
        col = jax.lax.broadcasted_iota(jnp.int32, x.shape, 1)
        y = jnp.where(col == 0, 0.0, x)
        y_norm = jnp.maximum(
            jnp.sqrt(jnp.sum(y * y, axis=-1, keepdims=True)), _MIN_NORM
        )
        theta = jnp.maximum(x[:, 0:1], 1.0 + _EPS)
        arc = jnp.log(
            theta + jnp.sqrt(jnp.maximum(theta * theta - 1.0, _MIN_NORM))
        )
        xt_ref[...] = arc * y / y_norm

    u = jnp.dot(a_ref[...], xt_ref[...], preferred_element_type=jnp.float32)
    col = jax.lax.broadcasted_iota(jnp.int32, u.shape, 1)
    us = jnp.where(col == 0, 0.0, u)
    n = jnp.maximum(
        jnp.sqrt(jnp.sum(us * us, axis=-1, keepdims=True)), _MIN_NORM
    )
    en = jnp.exp(n)
    sinh_n = 0.5 * (en - 1.0 / en)
    tail = sinh_n * us / n
    head = jnp.sqrt(
        jnp.maximum(1.0 + jnp.sum(tail * tail, axis=-1, keepdims=True), _EPS)
    )
    o_ref[...] = jnp.where(col == 0, head, tail)


def kernel(x, adj):
    n, d = x.shape
    m = adj.shape[0]
    bm = 200
    return pl.pallas_call(
        _lorentz_kernel,
        grid=(m // bm,),
        in_specs=[
            pl.BlockSpec((n, d), lambda i: (0, 0)),
            pl.BlockSpec((bm, n), lambda i: (i, 0)),
        ],
        out_specs=pl.BlockSpec((bm, d), lambda i: (i, 0)),
        out_shape=jax.ShapeDtypeStruct((m, d), x.dtype),
        scratch_shapes=[pltpu.VMEM((n, d), jnp.float32)],
        compiler_params=pltpu.CompilerParams(
            dimension_semantics=("arbitrary",),
            vmem_limit_bytes=100 * 1024 * 1024,
        ),
    )(x, adj)


# speedup vs baseline: 1.0172x; 1.0172x over previous
import jax
import jax.numpy as jnp
from jax.experimental import pallas as pl
from jax.experimental.pallas import tpu as pltpu

_MIN_NORM = 1e-15
_EPS = 1e-7


def _lorentz_kernel(x_ref, a1_ref, a2_ref, o_ref, xt_ref):
    @pl.when(pl.program_id(0) == 0)
    def _compute_tangent():
        x = x_ref[...]
        col = jax.lax.broadcasted_iota(jnp.int32, x.shape, 1)
        y = jnp.where(col == 0, 0.0, x)
        y_norm = jnp.maximum(
            jnp.sqrt(jnp.sum(y * y, axis=-1, keepdims=True)), _MIN_NORM
        )
        theta = jnp.maximum(x[:, 0:1], 1.0 + _EPS)
        arc = jnp.log(
            theta + jnp.sqrt(jnp.maximum(theta * theta - 1.0, _MIN_NORM))
        )
        xt_ref[...] = arc * y / y_norm

    xt = xt_ref[...]
    u = jnp.concatenate(
        [
            jnp.dot(a1_ref[...], xt, preferred_element_type=jnp.float32),
            jnp.dot(a2_ref[...], xt, preferred_element_type=jnp.float32),
        ],
        axis=0,
    )
    col = jax.lax.broadcasted_iota(jnp.int32, u.shape, 1)
    us = jnp.where(col == 0, 0.0, u)
    n = jnp.maximum(
        jnp.sqrt(jnp.sum(us * us, axis=-1, keepdims=True)), _MIN_NORM
    )
    en = jnp.exp(n)
    sinh_n = 0.5 * (en - 1.0 / en)
    tail = sinh_n * us / n
    head = jnp.sqrt(
        jnp.maximum(1.0 + jnp.sum(tail * tail, axis=-1, keepdims=True), _EPS)
    )
    o_ref[...] = jnp.where(col == 0, head, tail)


def kernel(x, adj):
    n, d = x.shape
    m = adj.shape[0]
    bm = 400
    return pl.pallas_call(
        _lorentz_kernel,
        grid=(m // bm,),
        in_specs=[
            pl.BlockSpec((n, d), lambda i: (0, 0)),
            pl.BlockSpec((bm // 2, n), lambda i: (2 * i, 0)),
            pl.BlockSpec((bm // 2, n), lambda i: (2 * i + 1, 0)),
        ],
        out_specs=pl.BlockSpec((bm, d), lambda i: (i, 0)),
        out_shape=jax.ShapeDtypeStruct((m, d), x.dtype),
        scratch_shapes=[pltpu.VMEM((n, d), jnp.float32)],
        compiler_params=pltpu.CompilerParams(
            dimension_semantics=("arbitrary",),
            vmem_limit_bytes=100 * 1024 * 1024,
        ),
    )(x, adj, adj)


# speedup vs baseline: 1.0381x; 1.0205x over previous
import jax
import jax.numpy as jnp
from jax.experimental import pallas as pl
from jax.experimental.pallas import tpu as pltpu

_MIN_NORM = 1e-15
_EPS = 1e-7


def _lorentz_kernel(x_ref, a_ref, o_ref, xt_ref):
    @pl.when(pl.program_id(0) == 0)
    def _compute_tangent():
        x = x_ref[...]
        col = jax.lax.broadcasted_iota(jnp.int32, x.shape, 1)
        y = jnp.where(col == 0, 0.0, x)
        y_norm = jnp.maximum(
            jnp.sqrt(jnp.sum(y * y, axis=-1, keepdims=True)), _MIN_NORM
        )
        theta = jnp.maximum(x[:, 0:1], 1.0 + _EPS)
        arc = jnp.log(
            theta + jnp.sqrt(jnp.maximum(theta * theta - 1.0, _MIN_NORM))
        )
        xt_ref[...] = arc * y / y_norm

    u = jnp.dot(a_ref[...], xt_ref[...], preferred_element_type=jnp.float32)
    col = jax.lax.broadcasted_iota(jnp.int32, u.shape, 1)
    us = jnp.where(col == 0, 0.0, u)
    n = jnp.maximum(
        jnp.sqrt(jnp.sum(us * us, axis=-1, keepdims=True)), _MIN_NORM
    )
    en = jnp.exp(n)
    sinh_n = 0.5 * (en - 1.0 / en)
    tail = sinh_n * us / n
    head = jnp.sqrt(
        jnp.maximum(1.0 + jnp.sum(tail * tail, axis=-1, keepdims=True), _EPS)
    )
    o_ref[...] = jnp.where(col == 0, head, tail)


def kernel(x, adj):
    n, d = x.shape
    m = adj.shape[0]
    bm = 400
    return pl.pallas_call(
        _lorentz_kernel,
        grid=(m // bm,),
        in_specs=[
            pl.BlockSpec((n, d), lambda i: (0, 0)),
            pl.BlockSpec((bm, n), lambda i: (i, 0)),
        ],
        out_specs=pl.BlockSpec((bm, d), lambda i: (i, 0)),
        out_shape=jax.ShapeDtypeStruct((m, d), x.dtype),
        scratch_shapes=[pltpu.VMEM((n, d), jnp.float32)],
        compiler_params=pltpu.CompilerParams(
            dimension_semantics=("arbitrary",),
            vmem_limit_bytes=100 * 1024 * 1024,
        ),
    )(x, adj)
